# Initial kernel scaffold; baseline (speedup 1.0000x reference)
import jax
import jax.numpy as jnp
from jax.experimental import pallas as pl


def kernel(x, edge_index, W1, b1, W2, b2, W3, b3):
    raise NotImplementedError("write your pallas kernel here")


# speedup vs baseline: 45.1853x; 45.1853x over previous
import functools

import jax
import jax.numpy as jnp
from jax import lax
from jax.experimental import pallas as pl
from jax.experimental.pallas import tpu as pltpu
from jax.experimental.pallas import tpu_sc as plsc

NC = 2
NS = 16
L = 16
WIN = 128
K = 8


def _row_prop_kernel(n_pad, total_win, feat_split):
    rps = n_pad // NS
    nzc = rps // WIN
    if feat_split:
        win_per_worker = total_win // NS
    else:
        win_per_worker = total_win // (NC * NS)
    steps = win_per_worker // K

    def body(tbl, srcw, dstw, out, acc, idx_s, idx_d, rows, zbuf,
             sem_s, sem_d, sem_g):
        c = lax.axis_index("c")
        s = lax.axis_index("s")
        r0 = s * rps

        def zb(i, carry):
            zbuf[i, :] = jnp.zeros((L,), jnp.float32)
            return carry
        lax.fori_loop(0, WIN, zb, 0)

        def zc(k, carry):
            pltpu.sync_copy(zbuf, acc.at[pl.ds(r0 + k * WIN, WIN), :])
            return carry
        lax.fori_loop(0, nzc, zc, 0)
        plsc.subcore_barrier()

        if feat_split:
            wd_first = s * win_per_worker
            ws_first = c * total_win + wd_first
        else:
            wd_first = (c * NS + s) * win_per_worker
            ws_first = wd_first

        def step(t, carry):
            cp_s = pltpu.async_copy(srcw.at[pl.ds(ws_first + t * K, K)],
                                    idx_s, sem_s)
            cp_d = pltpu.async_copy(dstw.at[pl.ds(wd_first + t * K, K)],
                                    idx_d, sem_d)
            cp_s.wait()
            cp_d.wait()
            gs = []
            for j in range(K):
                gs.append(pltpu.async_copy(tbl.at[idx_s.at[j]], rows.at[j], sem_g))
            for g in gs:
                g.wait()
            for j in range(K):
                pltpu.sync_copy(rows.at[j], acc.at[idx_d.at[j]], add=True)
            return carry
        lax.fori_loop(0, steps, step, 0)
        plsc.subcore_barrier()
        pltpu.sync_copy(acc.at[pl.ds(r0, rps), :],
                        out.at[pl.ds(c * n_pad + r0, rps), :])

    return pl.kernel(
        body,
        out_type=jax.ShapeDtypeStruct((NC * n_pad, L), jnp.float32),
        mesh=plsc.VectorSubcoreMesh(core_axis_name="c", subcore_axis_name="s"),
        compiler_params=pltpu.CompilerParams(use_tc_tiling_on_sc=False),
        scratch_types=[
            pltpu.VMEM_SHARED((n_pad, L), jnp.float32),
            pltpu.VMEM((K, WIN), jnp.int32),
            pltpu.VMEM((K, WIN), jnp.int32),
            pltpu.VMEM((K, WIN, L), jnp.float32),
            pltpu.VMEM((WIN, L), jnp.float32),
            pltpu.SemaphoreType.DMA,
            pltpu.SemaphoreType.DMA,
            pltpu.SemaphoreType.DMA,
        ],
    )


def _elem_prop_kernel(n_pad, total_win, gather):
    rps = n_pad // NS
    nzc = rps // WIN
    win_per_worker = total_win // (NC * NS)
    steps = win_per_worker // K

    def body(*refs):
        if gather:
            (tbl, srcw, dstw, out, acc, idx_s, idx_d, vals, zbuf,
             sem_s, sem_d, sem_g) = refs
        else:
            (dstw, out, acc, idx_d, vals, zbuf, sem_d) = refs
        c = lax.axis_index("c")
        s = lax.axis_index("s")
        r0 = s * rps

        def zb(i, carry):
            zbuf[pl.ds(i * L, L)] = jnp.zeros((L,), jnp.float32)
            return carry
        lax.fori_loop(0, WIN // L, zb, 0)

        def zc(k, carry):
            pltpu.sync_copy(zbuf, acc.at[pl.ds(r0 + k * WIN, WIN)])
            return carry
        lax.fori_loop(0, nzc, zc, 0)

        if not gather:
            for j in range(K):
                def ob(i, carry, j=j):
                    vals[j, pl.ds(i * L, L)] = jnp.ones((L,), jnp.float32)
                    return carry
                lax.fori_loop(0, WIN // L, ob, 0)
        plsc.subcore_barrier()

        w_first = (c * NS + s) * win_per_worker

        def step(t, carry):
            w0 = w_first + t * K
            if gather:
                cp_s = pltpu.async_copy(srcw.at[pl.ds(w0, K)], idx_s, sem_s)
            cp_d = pltpu.async_copy(dstw.at[pl.ds(w0, K)], idx_d, sem_d)
            if gather:
                cp_s.wait()
            cp_d.wait()
            if gather:
                gs = []
                for j in range(K):
                    gs.append(pltpu.async_copy(tbl.at[idx_s.at[j]], vals.at[j],
                                               sem_g))
                for g in gs:
                    g.wait()
            for j in range(K):
                pltpu.sync_copy(vals.at[j], acc.at[idx_d.at[j]], add=True)
            return carry
        lax.fori_loop(0, steps, step, 0)
        plsc.subcore_barrier()
        pltpu.sync_copy(acc.at[pl.ds(r0, rps)],
                        out.at[pl.ds(c * n_pad + r0, rps)])

    scratch = [
        pltpu.VMEM_SHARED((n_pad,), jnp.float32),
    ]
    if gather:
        scratch += [pltpu.VMEM((K, WIN), jnp.int32)]
    scratch += [
        pltpu.VMEM((K, WIN), jnp.int32),
        pltpu.VMEM((K, WIN), jnp.float32),
        pltpu.VMEM((WIN,), jnp.float32),
    ]
    if gather:
        scratch += [pltpu.SemaphoreType.DMA, pltpu.SemaphoreType.DMA,
                    pltpu.SemaphoreType.DMA]
    else:
        scratch += [pltpu.SemaphoreType.DMA]

    return pl.kernel(
        body,
        out_type=jax.ShapeDtypeStruct((NC * n_pad,), jnp.float32),
        mesh=plsc.VectorSubcoreMesh(core_axis_name="c", subcore_axis_name="s"),
        compiler_params=pltpu.CompilerParams(use_tc_tiling_on_sc=False),
        scratch_types=scratch,
    )


def _b1_body(p_ref, x_ref, dis_ref, g0_ref):
    deg = p_ref[0] + p_ref[1] + 1.0
    dis = lax.rsqrt(deg)
    dis_ref[...] = dis
    g0_ref[...] = x_ref[...] * dis


def _b2_body(dis_ref, p_ref, g0_ref, w1_ref, b1_ref, out_ref):
    dis = dis_ref[...]
    prop0 = dis * (p_ref[0] + p_ref[1] + g0_ref[...])
    h1 = jnp.dot(prop0, w1_ref[...], preferred_element_type=jnp.float32)
    h1 = jnp.maximum(h1 + b1_ref[...], 0.0)
    g1 = dis * h1
    out_ref[0] = g1[:, :L]
    out_ref[1] = g1[:, L:]


def _b3_body(dis_ref, p_ref, g1_ref, w2_ref, b2_ref, w3_ref, out_ref):
    dis = dis_ref[...]
    ph = dis * (p_ref[...] + g1_ref[...])
    h32 = jnp.concatenate([ph[0], ph[1]], axis=1)
    h2 = jnp.dot(h32, w2_ref[...], preferred_element_type=jnp.float32)
    h2 = jnp.maximum(h2 + b2_ref[...], 0.0)
    s = jnp.dot(h2, w3_ref[...], preferred_element_type=jnp.float32)
    out_ref[...] = dis * s


def _b4_body(dis_ref, p_ref, g2_ref, b3_ref, out_ref):
    t = dis_ref[...] * (p_ref[0] + p_ref[1] + g2_ref[...]) + b3_ref[...]
    out_ref[...] = jax.nn.sigmoid(t)


def kernel(x, edge_index, W1, b1, W2, b2, W3, b3):
    n = x.shape[0]
    e = edge_index.shape[1]
    n_pad = NS * WIN * -(-n // (NS * WIN))
    step_edges = NC * NS * K * WIN
    e_pad = step_edges * -(-e // step_edges)
    total_win = e_pad // WIN
    br = n_pad // NS
    grid = (n_pad // br,)

    src = edge_index[0].astype(jnp.int32)
    dst = edge_index[1].astype(jnp.int32)
    pad = n + (jnp.arange(e_pad - e, dtype=jnp.int32) % (n_pad - n))
    srcw = jnp.concatenate([src, pad]).reshape(total_win, WIN)
    dstw = jnp.concatenate([dst, pad]).reshape(total_win, WIN)
    src2w = jnp.concatenate([srcw, srcw + n_pad], axis=0)
    x_pad = jnp.pad(x, ((0, n_pad - n), (0, 0)))

    f32 = jnp.float32
    sds = jax.ShapeDtypeStruct

    degp = _elem_prop_kernel(n_pad, total_win, gather=False)(dstw)
    degp = degp.reshape(NC, n_pad, 1)

    dis, g0 = pl.pallas_call(
        _b1_body,
        grid=grid,
        in_specs=[
            pl.BlockSpec((NC, br, 1), lambda i: (0, i, 0)),
            pl.BlockSpec((br, L), lambda i: (i, 0)),
        ],
        out_specs=[
            pl.BlockSpec((br, 1), lambda i: (i, 0)),
            pl.BlockSpec((br, L), lambda i: (i, 0)),
        ],
        out_shape=[sds((n_pad, 1), f32), sds((n_pad, L), f32)],
    )(degp, x_pad)

    p0 = _row_prop_kernel(n_pad, total_win, feat_split=False)(g0, srcw, dstw)
    p0 = p0.reshape(NC, n_pad, L)

    g1h = pl.pallas_call(
        _b2_body,
        grid=grid,
        in_specs=[
            pl.BlockSpec((br, 1), lambda i: (i, 0)),
            pl.BlockSpec((NC, br, L), lambda i: (0, i, 0)),
            pl.BlockSpec((br, L), lambda i: (i, 0)),
            pl.BlockSpec((L, 2 * L), lambda i: (0, 0)),
            pl.BlockSpec((1, 2 * L), lambda i: (0, 0)),
        ],
        out_specs=pl.BlockSpec((NC, br, L), lambda i: (0, i, 0)),
        out_shape=sds((NC, n_pad, L), f32),
    )(dis, p0, g0, W1, b1.reshape(1, 2 * L))

    p1 = _row_prop_kernel(n_pad, total_win, feat_split=True)(
        g1h.reshape(NC * n_pad, L), src2w, dstw)
    p1 = p1.reshape(NC, n_pad, L)

    g2 = pl.pallas_call(
        _b3_body,
        grid=grid,
        in_specs=[
            pl.BlockSpec((br, 1), lambda i: (i, 0)),
            pl.BlockSpec((NC, br, L), lambda i: (0, i, 0)),
            pl.BlockSpec((NC, br, L), lambda i: (0, i, 0)),
            pl.BlockSpec((2 * L, 4 * L), lambda i: (0, 0)),
            pl.BlockSpec((1, 4 * L), lambda i: (0, 0)),
            pl.BlockSpec((4 * L, 1), lambda i: (0, 0)),
        ],
        out_specs=pl.BlockSpec((br, 1), lambda i: (i, 0)),
        out_shape=sds((n_pad, 1), f32),
    )(dis, p1, g1h, W2, b2.reshape(1, 4 * L), W3)

    p2 = _elem_prop_kernel(n_pad, total_win, gather=True)(
        g2.reshape(n_pad), srcw, dstw)
    p2 = p2.reshape(NC, n_pad, 1)

    out = pl.pallas_call(
        _b4_body,
        grid=grid,
        in_specs=[
            pl.BlockSpec((br, 1), lambda i: (i, 0)),
            pl.BlockSpec((NC, br, 1), lambda i: (0, i, 0)),
            pl.BlockSpec((br, 1), lambda i: (i, 0)),
            pl.BlockSpec((1, 1), lambda i: (0, 0)),
        ],
        out_specs=pl.BlockSpec((br, 1), lambda i: (i, 0)),
        out_shape=sds((n_pad, 1), f32),
    )(dis, p2, g2, b3.reshape(1, 1))

    return out[:n]


# speedup vs baseline: 54.1901x; 1.1993x over previous
import jax
import jax.numpy as jnp
from jax import lax
from jax.experimental import pallas as pl
from jax.experimental.pallas import tpu as pltpu
from jax.experimental.pallas import tpu_sc as plsc

NC = 2
NS = 16
L = 16
WIN = 128
K = 4


def _row_prop_kernel(n_pad, total_win, feat_split):
    rps = n_pad // NS
    nzc = rps // WIN
    if feat_split:
        win_per_worker = total_win // NS
    else:
        win_per_worker = total_win // (NC * NS)
    steps = win_per_worker // K
    assert steps % 2 == 0

    def body(tbl, srcw, dstw, out, acc,
             idx_s0, idx_s1, idx_d0, idx_d1, rows0, rows1, zbuf,
             sem_s0, sem_s1, sem_d0, sem_d1, sem_g0, sem_g1):
        idx_s = (idx_s0, idx_s1)
        idx_d = (idx_d0, idx_d1)
        rows = (rows0, rows1)
        sem_s = (sem_s0, sem_s1)
        sem_d = (sem_d0, sem_d1)
        sem_g = (sem_g0, sem_g1)
        c = lax.axis_index("c")
        s = lax.axis_index("s")
        r0 = s * rps

        def zb(i, carry):
            zbuf[i, :] = jnp.zeros((L,), jnp.float32)
            return carry
        lax.fori_loop(0, WIN, zb, 0)

        def zc(k, carry):
            pltpu.sync_copy(zbuf, acc.at[pl.ds(r0 + k * WIN, WIN), :])
            return carry
        lax.fori_loop(0, nzc, zc, 0)
        plsc.subcore_barrier()

        if feat_split:
            w_first = s * win_per_worker
        else:
            w_first = (c * NS + s) * win_per_worker

        def stage(b, ss):
            w0 = w_first + ss * K
            pltpu.async_copy(srcw.at[pl.ds(w0, K)], idx_s[b], sem_s[b])
            pltpu.async_copy(dstw.at[pl.ds(w0, K)], idx_d[b], sem_d[b])

        def wait_stage(b):
            pltpu.make_async_copy(srcw.at[pl.ds(0, K)], idx_s[b], sem_s[b]).wait()
            pltpu.make_async_copy(dstw.at[pl.ds(0, K)], idx_d[b], sem_d[b]).wait()

        def fire(b):
            if feat_split:
                off = c * n_pad
                for j in range(K):
                    for i in range(WIN // L):
                        sl = pl.ds(i * L, L)
                        idx_s[b][j, sl] = idx_s[b][j, sl] + off
            for j in range(K):
                pltpu.async_copy(tbl.at[idx_s[b].at[j]], rows[b].at[j], sem_g[b])

        def wait_fire(b):
            for j in range(K):
                pltpu.make_async_copy(tbl.at[idx_s[b].at[j]], rows[b].at[j],
                                      sem_g[b]).wait()

        def scatter(b):
            for j in range(K):
                pltpu.sync_copy(rows[b].at[j], acc.at[idx_d[b].at[j]], add=True)

        stage(0, 0)
        wait_stage(0)
        fire(0)
        stage(1, 1)

        def pair(t, carry):
            ss0 = 2 * t
            for b in (0, 1):
                nb = 1 - b
                wait_stage(nb)
                fire(nb)
                wait_fire(b)
                scatter(b)
                stage(b, ss0 + b + 2)
            return carry
        lax.fori_loop(0, steps // 2, pair, 0)
        wait_fire(0)
        wait_stage(1)

        plsc.subcore_barrier()
        pltpu.sync_copy(acc.at[pl.ds(r0, rps), :],
                        out.at[pl.ds(c * n_pad + r0, rps), :])

    return pl.kernel(
        body,
        out_type=jax.ShapeDtypeStruct((NC * n_pad, L), jnp.float32),
        mesh=plsc.VectorSubcoreMesh(core_axis_name="c", subcore_axis_name="s"),
        compiler_params=pltpu.CompilerParams(use_tc_tiling_on_sc=False),
        scratch_types=[
            pltpu.VMEM_SHARED((n_pad, L), jnp.float32),
            pltpu.VMEM((K, WIN), jnp.int32),
            pltpu.VMEM((K, WIN), jnp.int32),
            pltpu.VMEM((K, WIN), jnp.int32),
            pltpu.VMEM((K, WIN), jnp.int32),
            pltpu.VMEM((K, WIN, L), jnp.float32),
            pltpu.VMEM((K, WIN, L), jnp.float32),
            pltpu.VMEM((WIN, L), jnp.float32),
        ] + [pltpu.SemaphoreType.DMA] * 6,
    )


def _elem_prop_kernel(n_pad, total_win, gather):
    rps = n_pad // NS
    nzc = rps // WIN
    win_per_worker = total_win // (NC * NS)
    steps = win_per_worker // K
    assert steps % 2 == 0

    def body(*refs):
        if gather:
            (tbl, srcw, dstw, out, acc,
             idx_s0, idx_s1, idx_d0, idx_d1, vals0, vals1, zbuf,
             sem_s0, sem_s1, sem_d0, sem_d1, sem_g0, sem_g1) = refs
            idx_s = (idx_s0, idx_s1)
            vals = (vals0, vals1)
            sem_s = (sem_s0, sem_s1)
            sem_g = (sem_g0, sem_g1)
        else:
            (dstw, out, acc, idx_d0, idx_d1, ones, zbuf,
             sem_d0, sem_d1) = refs
            vals = (ones, ones)
        idx_d = (idx_d0, idx_d1)
        sem_d = (sem_d0, sem_d1)
        c = lax.axis_index("c")
        s = lax.axis_index("s")
        r0 = s * rps

        def zb(i, carry):
            zbuf[pl.ds(i * L, L)] = jnp.zeros((L,), jnp.float32)
            return carry
        lax.fori_loop(0, WIN // L, zb, 0)

        def zc(k, carry):
            pltpu.sync_copy(zbuf, acc.at[pl.ds(r0 + k * WIN, WIN)])
            return carry
        lax.fori_loop(0, nzc, zc, 0)

        if not gather:
            for j in range(K):
                def ob(i, carry, j=j):
                    vals[0][j, pl.ds(i * L, L)] = jnp.ones((L,), jnp.float32)
                    return carry
                lax.fori_loop(0, WIN // L, ob, 0)
        plsc.subcore_barrier()

        w_first = (c * NS + s) * win_per_worker

        def stage(b, ss):
            w0 = w_first + ss * K
            if gather:
                pltpu.async_copy(srcw.at[pl.ds(w0, K)], idx_s[b], sem_s[b])
            pltpu.async_copy(dstw.at[pl.ds(w0, K)], idx_d[b], sem_d[b])

        def wait_stage(b):
            if gather:
                pltpu.make_async_copy(srcw.at[pl.ds(0, K)], idx_s[b],
                                      sem_s[b]).wait()
            pltpu.make_async_copy(dstw.at[pl.ds(0, K)], idx_d[b], sem_d[b]).wait()

        def fire(b):
            if gather:
                for j in range(K):
                    pltpu.async_copy(tbl.at[idx_s[b].at[j]], vals[b].at[j],
                                     sem_g[b])

        def wait_fire(b):
            if gather:
                for j in range(K):
                    pltpu.make_async_copy(tbl.at[idx_s[b].at[j]], vals[b].at[j],
                                          sem_g[b]).wait()

        def scatter(b):
            for j in range(K):
                pltpu.sync_copy(vals[b].at[j], acc.at[idx_d[b].at[j]], add=True)

        stage(0, 0)
        wait_stage(0)
        fire(0)
        stage(1, 1)

        def pair(t, carry):
            ss0 = 2 * t
            for b in (0, 1):
                nb = 1 - b
                wait_stage(nb)
                fire(nb)
                wait_fire(b)
                scatter(b)
                stage(b, ss0 + b + 2)
            return carry
        lax.fori_loop(0, steps // 2, pair, 0)
        wait_fire(0)
        wait_stage(1)

        plsc.subcore_barrier()
        pltpu.sync_copy(acc.at[pl.ds(r0, rps)],
                        out.at[pl.ds(c * n_pad + r0, rps)])

    scratch = [pltpu.VMEM_SHARED((n_pad,), jnp.float32)]
    if gather:
        scratch += [pltpu.VMEM((K, WIN), jnp.int32)] * 2
    scratch += [pltpu.VMEM((K, WIN), jnp.int32)] * 2
    if gather:
        scratch += [pltpu.VMEM((K, WIN), jnp.float32)] * 2
    else:
        scratch += [pltpu.VMEM((K, WIN), jnp.float32)]
    scratch += [pltpu.VMEM((WIN,), jnp.float32)]
    scratch += [pltpu.SemaphoreType.DMA] * (6 if gather else 2)

    return pl.kernel(
        body,
        out_type=jax.ShapeDtypeStruct((NC * n_pad,), jnp.float32),
        mesh=plsc.VectorSubcoreMesh(core_axis_name="c", subcore_axis_name="s"),
        compiler_params=pltpu.CompilerParams(use_tc_tiling_on_sc=False),
        scratch_types=scratch,
    )


def kernel(x, edge_index, W1, b1, W2, b2, W3, b3):
    n = x.shape[0]
    e = edge_index.shape[1]
    n_pad = NS * WIN * -(-n // (NS * WIN))
    step_edges = NC * NS * 2 * K * WIN
    e_pad = step_edges * -(-e // step_edges)
    total_win = e_pad // WIN
    s_rows = total_win + 2 * K
    br = n_pad // NS
    grid = (n_pad // br,)

    src = edge_index[0].astype(jnp.int32)
    dst = edge_index[1].astype(jnp.int32)
    npad_extra = s_rows * WIN - e
    pad = n + (jnp.arange(npad_extra, dtype=jnp.int32) % (n_pad - n))
    srcw = jnp.concatenate([src, pad]).reshape(s_rows, WIN)
    dstw = jnp.concatenate([dst, pad]).reshape(s_rows, WIN)
    x_pad = jnp.pad(x, ((0, n_pad - n), (0, 0)))

    f32 = jnp.float32
    sds = jax.ShapeDtypeStruct
    nb = n_pad // br

    halves = (pl.BlockSpec((br, 1), lambda i: (i, 0)),
              pl.BlockSpec((br, 1), lambda i: (nb + i, 0)))
    rhalf = (pl.BlockSpec((br, L), lambda i: (i, 0)),
             pl.BlockSpec((br, L), lambda i: (nb + i, 0)))

    degp = _elem_prop_kernel(n_pad, total_win, gather=False)(dstw)
    degp = degp.reshape(NC * n_pad, 1)

    def b1_body(pa_ref, pb_ref, x_ref, dis_ref, g0_ref):
        deg = pa_ref[...] + pb_ref[...] + 1.0
        dis = lax.rsqrt(deg)
        dis_ref[...] = dis
        g0_ref[...] = x_ref[...] * dis

    dis, g0 = pl.pallas_call(
        b1_body,
        grid=grid,
        in_specs=[*halves, pl.BlockSpec((br, L), lambda i: (i, 0))],
        out_specs=[
            pl.BlockSpec((br, 1), lambda i: (i, 0)),
            pl.BlockSpec((br, L), lambda i: (i, 0)),
        ],
        out_shape=[sds((n_pad, 1), f32), sds((n_pad, L), f32)],
    )(degp, degp, x_pad)

    p0 = _row_prop_kernel(n_pad, total_win, feat_split=False)(g0, srcw, dstw)

    def b2_body(dis_ref, pa_ref, pb_ref, g0_ref, w1_ref, b1_ref, out_ref):
        sel = pl.program_id(0) >= nb
        dis = dis_ref[...]
        prop0 = dis * (pa_ref[...] + pb_ref[...] + g0_ref[...])
        h1 = jnp.dot(prop0, w1_ref[...], preferred_element_type=jnp.float32)
        h1 = jnp.maximum(h1 + b1_ref[...], 0.0)
        g1 = dis * h1
        out_ref[...] = jnp.where(sel, g1[:, L:], g1[:, :L])

    g1h = pl.pallas_call(
        b2_body,
        grid=(2 * nb,),
        in_specs=[
            pl.BlockSpec((br, 1), lambda i: (i % nb, 0)),
            pl.BlockSpec((br, L), lambda i: (i % nb, 0)),
            pl.BlockSpec((br, L), lambda i: (nb + i % nb, 0)),
            pl.BlockSpec((br, L), lambda i: (i % nb, 0)),
            pl.BlockSpec((L, 2 * L), lambda i: (0, 0)),
            pl.BlockSpec((1, 2 * L), lambda i: (0, 0)),
        ],
        out_specs=pl.BlockSpec((br, L), lambda i: (i, 0)),
        out_shape=sds((NC * n_pad, L), f32),
    )(dis, p0, p0, g0, W1, b1.reshape(1, 2 * L))

    p1 = _row_prop_kernel(n_pad, total_win, feat_split=True)(g1h, srcw, dstw)

    def b3_body(dis_ref, pa_ref, pb_ref, ga_ref, gb_ref, w2_ref, b2_ref,
                w3_ref, out_ref):
        dis = dis_ref[...]
        h32 = jnp.concatenate([dis * (pa_ref[...] + ga_ref[...]),
                               dis * (pb_ref[...] + gb_ref[...])], axis=1)
        h2 = jnp.dot(h32, w2_ref[...], preferred_element_type=jnp.float32)
        h2 = jnp.maximum(h2 + b2_ref[...], 0.0)
        s = jnp.dot(h2, w3_ref[...], preferred_element_type=jnp.float32)
        out_ref[...] = dis * s

    g2 = pl.pallas_call(
        b3_body,
        grid=grid,
        in_specs=[
            pl.BlockSpec((br, 1), lambda i: (i, 0)),
            *rhalf,
            *rhalf,
            pl.BlockSpec((2 * L, 4 * L), lambda i: (0, 0)),
            pl.BlockSpec((1, 4 * L), lambda i: (0, 0)),
            pl.BlockSpec((4 * L, 1), lambda i: (0, 0)),
        ],
        out_specs=pl.BlockSpec((br, 1), lambda i: (i, 0)),
        out_shape=sds((n_pad, 1), f32),
    )(dis, p1, p1, g1h, g1h, W2, b2.reshape(1, 4 * L), W3)

    p2 = _elem_prop_kernel(n_pad, total_win, gather=True)(
        g2.reshape(n_pad), srcw, dstw)
    p2 = p2.reshape(NC * n_pad, 1)

    def b4_body(dis_ref, pa_ref, pb_ref, g2_ref, b3_ref, out_ref):
        t = pa_ref[...] + pb_ref[...] + g2_ref[...]
        t = dis_ref[...] * t + b3_ref[...]
        out_ref[...] = jax.nn.sigmoid(t)

    out = pl.pallas_call(
        b4_body,
        grid=grid,
        in_specs=[
            pl.BlockSpec((br, 1), lambda i: (i, 0)),
            *halves,
            pl.BlockSpec((br, 1), lambda i: (i, 0)),
            pl.BlockSpec((1, 1), lambda i: (0, 0)),
        ],
        out_specs=pl.BlockSpec((br, 1), lambda i: (i, 0)),
        out_shape=sds((n_pad, 1), f32),
    )(dis, p2, p2, g2, b3.reshape(1, 1))

    return out[:n]


# speedup vs baseline: 57.5025x; 1.0611x over previous
import jax
import jax.numpy as jnp
from jax import lax
from jax.experimental import pallas as pl
from jax.experimental.pallas import tpu as pltpu
from jax.experimental.pallas import tpu_sc as plsc

NC = 2
NS = 16
L = 16
WIN = 128
K = 4


def _row_prop_kernel(n_pad, total_win, feat_split):
    rps = n_pad // NS
    nzc = rps // WIN
    if feat_split:
        win_per_worker = total_win // NS
    else:
        win_per_worker = total_win // (NC * NS)
    steps = win_per_worker // K
    assert steps % 2 == 0

    def body(tbl, srcw, dstw, out, acc,
             idx_s0, idx_s1, idx_d0, idx_d1, rows0, rows1, zbuf,
             sem_s0, sem_s1, sem_d0, sem_d1, sem_g0, sem_g1, sem_c0, sem_c1):
        idx_s = (idx_s0, idx_s1)
        idx_d = (idx_d0, idx_d1)
        rows = (rows0, rows1)
        sem_s = (sem_s0, sem_s1)
        sem_d = (sem_d0, sem_d1)
        sem_g = (sem_g0, sem_g1)
        sem_c = (sem_c0, sem_c1)
        c = lax.axis_index("c")
        s = lax.axis_index("s")
        r0 = s * rps

        def zb(i, carry):
            zbuf[i, :] = jnp.zeros((L,), jnp.float32)
            return carry
        lax.fori_loop(0, WIN, zb, 0)

        def zc(k, carry):
            pltpu.sync_copy(zbuf, acc.at[pl.ds(r0 + k * WIN, WIN), :])
            return carry
        lax.fori_loop(0, nzc, zc, 0)
        plsc.subcore_barrier()

        if feat_split:
            w_first = s * win_per_worker
        else:
            w_first = (c * NS + s) * win_per_worker

        def stage(b, ss):
            w0 = w_first + ss * K
            pltpu.async_copy(srcw.at[pl.ds(w0, K)], idx_s[b], sem_s[b])
            pltpu.async_copy(dstw.at[pl.ds(w0, K)], idx_d[b], sem_d[b])

        def wait_stage(b):
            pltpu.make_async_copy(srcw.at[pl.ds(0, K)], idx_s[b], sem_s[b]).wait()
            pltpu.make_async_copy(dstw.at[pl.ds(0, K)], idx_d[b], sem_d[b]).wait()

        def fire(b):
            if feat_split:
                off = c * n_pad
                for j in range(K):
                    for i in range(WIN // L):
                        sl = pl.ds(i * L, L)
                        idx_s[b][j, sl] = idx_s[b][j, sl] + off
            for j in range(K):
                pltpu.async_copy(tbl.at[idx_s[b].at[j]], rows[b].at[j], sem_g[b])

        def wait_fire(b):
            for j in range(K):
                pltpu.make_async_copy(tbl.at[idx_s[b].at[j]], rows[b].at[j],
                                      sem_g[b]).wait()

        def scatter(b):
            for j in range(K):
                pltpu.async_copy(rows[b].at[j], acc.at[idx_d[b].at[j]],
                                 sem_c[b], add=True)
            for j in range(K):
                pltpu.make_async_copy(rows[b].at[j], acc.at[idx_d[b].at[j]],
                                      sem_c[b]).wait()

        stage(0, 0)
        wait_stage(0)
        fire(0)
        stage(1, 1)

        def pair(t, carry):
            ss0 = 2 * t
            for b in (0, 1):
                nb = 1 - b
                wait_stage(nb)
                fire(nb)
                wait_fire(b)
                scatter(b)
                stage(b, ss0 + b + 2)
            return carry
        lax.fori_loop(0, steps // 2, pair, 0)
        wait_fire(0)
        wait_stage(1)

        plsc.subcore_barrier()
        pltpu.sync_copy(acc.at[pl.ds(r0, rps), :],
                        out.at[pl.ds(c * n_pad + r0, rps), :])

    return pl.kernel(
        body,
        out_type=jax.ShapeDtypeStruct((NC * n_pad, L), jnp.float32),
        mesh=plsc.VectorSubcoreMesh(core_axis_name="c", subcore_axis_name="s"),
        compiler_params=pltpu.CompilerParams(use_tc_tiling_on_sc=False),
        scratch_types=[
            pltpu.VMEM_SHARED((n_pad, L), jnp.float32),
            pltpu.VMEM((K, WIN), jnp.int32),
            pltpu.VMEM((K, WIN), jnp.int32),
            pltpu.VMEM((K, WIN), jnp.int32),
            pltpu.VMEM((K, WIN), jnp.int32),
            pltpu.VMEM((K, WIN, L), jnp.float32),
            pltpu.VMEM((K, WIN, L), jnp.float32),
            pltpu.VMEM((WIN, L), jnp.float32),
        ] + [pltpu.SemaphoreType.DMA] * 8,
    )


def _elem_prop_kernel(n_pad, total_win, gather):
    rps = n_pad // NS
    nzc = rps // WIN
    win_per_worker = total_win // (NC * NS)
    steps = win_per_worker // K
    assert steps % 2 == 0

    def body(*refs):
        if gather:
            (tbl, srcw, dstw, out, acc,
             idx_s0, idx_s1, idx_d0, idx_d1, vals0, vals1, zbuf,
             sem_s0, sem_s1, sem_d0, sem_d1, sem_g0, sem_g1,
             sem_c0, sem_c1) = refs
            idx_s = (idx_s0, idx_s1)
            vals = (vals0, vals1)
            sem_s = (sem_s0, sem_s1)
            sem_g = (sem_g0, sem_g1)
        else:
            (dstw, out, acc, idx_d0, idx_d1, ones, zbuf,
             sem_d0, sem_d1, sem_c0, sem_c1) = refs
            vals = (ones, ones)
        idx_d = (idx_d0, idx_d1)
        sem_d = (sem_d0, sem_d1)
        sem_c = (sem_c0, sem_c1)
        c = lax.axis_index("c")
        s = lax.axis_index("s")
        r0 = s * rps

        def zb(i, carry):
            zbuf[pl.ds(i * L, L)] = jnp.zeros((L,), jnp.float32)
            return carry
        lax.fori_loop(0, WIN // L, zb, 0)

        def zc(k, carry):
            pltpu.sync_copy(zbuf, acc.at[pl.ds(r0 + k * WIN, WIN)])
            return carry
        lax.fori_loop(0, nzc, zc, 0)

        if not gather:
            for j in range(K):
                def ob(i, carry, j=j):
                    vals[0][j, pl.ds(i * L, L)] = jnp.ones((L,), jnp.float32)
                    return carry
                lax.fori_loop(0, WIN // L, ob, 0)
        plsc.subcore_barrier()

        w_first = (c * NS + s) * win_per_worker

        def stage(b, ss):
            w0 = w_first + ss * K
            if gather:
                pltpu.async_copy(srcw.at[pl.ds(w0, K)], idx_s[b], sem_s[b])
            pltpu.async_copy(dstw.at[pl.ds(w0, K)], idx_d[b], sem_d[b])

        def wait_stage(b):
            if gather:
                pltpu.make_async_copy(srcw.at[pl.ds(0, K)], idx_s[b],
                                      sem_s[b]).wait()
            pltpu.make_async_copy(dstw.at[pl.ds(0, K)], idx_d[b], sem_d[b]).wait()

        def fire(b):
            if gather:
                for j in range(K):
                    pltpu.async_copy(tbl.at[idx_s[b].at[j]], vals[b].at[j],
                                     sem_g[b])

        def wait_fire(b):
            if gather:
                for j in range(K):
                    pltpu.make_async_copy(tbl.at[idx_s[b].at[j]], vals[b].at[j],
                                          sem_g[b]).wait()

        def scatter(b):
            for j in range(K):
                pltpu.async_copy(vals[b].at[j], acc.at[idx_d[b].at[j]],
                                 sem_c[b], add=True)
            for j in range(K):
                pltpu.make_async_copy(vals[b].at[j], acc.at[idx_d[b].at[j]],
                                      sem_c[b]).wait()

        stage(0, 0)
        wait_stage(0)
        fire(0)
        stage(1, 1)

        def pair(t, carry):
            ss0 = 2 * t
            for b in (0, 1):
                nb = 1 - b
                wait_stage(nb)
                fire(nb)
                wait_fire(b)
                scatter(b)
                stage(b, ss0 + b + 2)
            return carry
        lax.fori_loop(0, steps // 2, pair, 0)
        wait_fire(0)
        wait_stage(1)

        plsc.subcore_barrier()
        pltpu.sync_copy(acc.at[pl.ds(r0, rps)],
                        out.at[pl.ds(c * n_pad + r0, rps)])

    scratch = [pltpu.VMEM_SHARED((n_pad,), jnp.float32)]
    if gather:
        scratch += [pltpu.VMEM((K, WIN), jnp.int32)] * 2
    scratch += [pltpu.VMEM((K, WIN), jnp.int32)] * 2
    if gather:
        scratch += [pltpu.VMEM((K, WIN), jnp.float32)] * 2
    else:
        scratch += [pltpu.VMEM((K, WIN), jnp.float32)]
    scratch += [pltpu.VMEM((WIN,), jnp.float32)]
    scratch += [pltpu.SemaphoreType.DMA] * (8 if gather else 4)

    return pl.kernel(
        body,
        out_type=jax.ShapeDtypeStruct((NC * n_pad,), jnp.float32),
        mesh=plsc.VectorSubcoreMesh(core_axis_name="c", subcore_axis_name="s"),
        compiler_params=pltpu.CompilerParams(use_tc_tiling_on_sc=False),
        scratch_types=scratch,
    )


def kernel(x, edge_index, W1, b1, W2, b2, W3, b3):
    n = x.shape[0]
    e = edge_index.shape[1]
    n_pad = NS * WIN * -(-n // (NS * WIN))
    step_edges = NC * NS * 2 * K * WIN
    e_pad = step_edges * -(-e // step_edges)
    total_win = e_pad // WIN
    s_rows = total_win + 2 * K
    br = n_pad // NS
    grid = (n_pad // br,)

    src = edge_index[0].astype(jnp.int32)
    dst = edge_index[1].astype(jnp.int32)
    npad_extra = s_rows * WIN - e
    pad = n + (jnp.arange(npad_extra, dtype=jnp.int32) % (n_pad - n))
    srcw = jnp.concatenate([src, pad]).reshape(s_rows, WIN)
    dstw = jnp.concatenate([dst, pad]).reshape(s_rows, WIN)

    f32 = jnp.float32
    sds = jax.ShapeDtypeStruct
    nb = n_pad // br

    halves = (pl.BlockSpec((br, 1), lambda i: (i, 0)),
              pl.BlockSpec((br, 1), lambda i: (nb + i, 0)))
    rhalf = (pl.BlockSpec((br, L), lambda i: (i, 0)),
             pl.BlockSpec((br, L), lambda i: (nb + i, 0)))

    degp = _elem_prop_kernel(n_pad, total_win, gather=False)(dstw)
    degp = degp.reshape(NC * n_pad, 1)

    def b1_body(pa_ref, pb_ref, x_ref, dis_ref, g0_ref):
        deg = pa_ref[...] + pb_ref[...] + 1.0
        dis = lax.rsqrt(deg)
        dis_ref[...] = dis
        g0_ref[...] = x_ref[...] * dis

    dis, g0 = pl.pallas_call(
        b1_body,
        grid=grid,
        in_specs=[*halves, pl.BlockSpec((br, L), lambda i: (i, 0))],
        out_specs=[
            pl.BlockSpec((br, 1), lambda i: (i, 0)),
            pl.BlockSpec((br, L), lambda i: (i, 0)),
        ],
        out_shape=[sds((n_pad, 1), f32), sds((n_pad, L), f32)],
    )(degp, degp, x)

    p0 = _row_prop_kernel(n_pad, total_win, feat_split=False)(g0, srcw, dstw)

    def b2_body(dis_ref, pa_ref, pb_ref, g0_ref, w1_ref, b1_ref, out_ref):
        sel = pl.program_id(0) >= nb
        dis = dis_ref[...]
        prop0 = dis * (pa_ref[...] + pb_ref[...] + g0_ref[...])
        h1 = jnp.dot(prop0, w1_ref[...], preferred_element_type=jnp.float32)
        h1 = jnp.maximum(h1 + b1_ref[...], 0.0)
        g1 = dis * h1
        out_ref[...] = jnp.where(sel, g1[:, L:], g1[:, :L])

    g1h = pl.pallas_call(
        b2_body,
        grid=(2 * nb,),
        in_specs=[
            pl.BlockSpec((br, 1), lambda i: (i % nb, 0)),
            pl.BlockSpec((br, L), lambda i: (i % nb, 0)),
            pl.BlockSpec((br, L), lambda i: (nb + i % nb, 0)),
            pl.BlockSpec((br, L), lambda i: (i % nb, 0)),
            pl.BlockSpec((L, 2 * L), lambda i: (0, 0)),
            pl.BlockSpec((1, 2 * L), lambda i: (0, 0)),
        ],
        out_specs=pl.BlockSpec((br, L), lambda i: (i, 0)),
        out_shape=sds((NC * n_pad, L), f32),
    )(dis, p0, p0, g0, W1, b1.reshape(1, 2 * L))

    p1 = _row_prop_kernel(n_pad, total_win, feat_split=True)(g1h, srcw, dstw)

    def b3_body(dis_ref, pa_ref, pb_ref, ga_ref, gb_ref, w2_ref, b2_ref,
                w3_ref, out_ref):
        dis = dis_ref[...]
        h32 = jnp.concatenate([dis * (pa_ref[...] + ga_ref[...]),
                               dis * (pb_ref[...] + gb_ref[...])], axis=1)
        h2 = jnp.dot(h32, w2_ref[...], preferred_element_type=jnp.float32)
        h2 = jnp.maximum(h2 + b2_ref[...], 0.0)
        s = jnp.dot(h2, w3_ref[...], preferred_element_type=jnp.float32)
        out_ref[...] = dis * s

    g2 = pl.pallas_call(
        b3_body,
        grid=grid,
        in_specs=[
            pl.BlockSpec((br, 1), lambda i: (i, 0)),
            *rhalf,
            *rhalf,
            pl.BlockSpec((2 * L, 4 * L), lambda i: (0, 0)),
            pl.BlockSpec((1, 4 * L), lambda i: (0, 0)),
            pl.BlockSpec((4 * L, 1), lambda i: (0, 0)),
        ],
        out_specs=pl.BlockSpec((br, 1), lambda i: (i, 0)),
        out_shape=sds((n_pad, 1), f32),
    )(dis, p1, p1, g1h, g1h, W2, b2.reshape(1, 4 * L), W3)

    p2 = _elem_prop_kernel(n_pad, total_win, gather=True)(
        g2.reshape(n_pad), srcw, dstw)
    p2 = p2.reshape(NC * n_pad, 1)

    def b4_body(dis_ref, pa_ref, pb_ref, g2_ref, b3_ref, out_ref):
        t = pa_ref[...] + pb_ref[...] + g2_ref[...]
        t = dis_ref[...] * t + b3_ref[0, 0]
        out_ref[...] = jax.nn.sigmoid(t)

    out = pl.pallas_call(
        b4_body,
        grid=grid,
        in_specs=[
            pl.BlockSpec((br, 1), lambda i: (i, 0)),
            *halves,
            pl.BlockSpec((br, 1), lambda i: (i, 0)),
            pl.BlockSpec((1, 1), lambda i: (0, 0)),
        ],
        out_specs=pl.BlockSpec((br, 1), lambda i: (i, 0)),
        out_shape=sds((n_pad, 1), f32),
    )(dis, p2, p2, g2, b3.reshape(1, 1))

    return out[:n]


# speedup vs baseline: 62.5958x; 1.0886x over previous
import jax
import jax.numpy as jnp
from jax import lax
from jax.experimental import pallas as pl
from jax.experimental.pallas import tpu as pltpu
from jax.experimental.pallas import tpu_sc as plsc

NC = 2
NS = 16
L = 16
WIN = 128
K = 6


def _row_prop_kernel(n_pad, total_win, feat_split):
    rps = n_pad // NS
    nzc = rps // WIN
    if feat_split:
        win_per_worker = total_win // NS
    else:
        win_per_worker = total_win // (NC * NS)
    steps = win_per_worker // K
    assert steps % 2 == 0

    def body(tbl, srcw, dstw, out, acc,
             idx_s0, idx_s1, idx_d0, idx_d1, rows0, rows1, zbuf,
             sem_s0, sem_s1, sem_d0, sem_d1, sem_g0, sem_g1, sem_c0, sem_c1):
        idx_s = (idx_s0, idx_s1)
        idx_d = (idx_d0, idx_d1)
        rows = (rows0, rows1)
        sem_s = (sem_s0, sem_s1)
        sem_d = (sem_d0, sem_d1)
        sem_g = (sem_g0, sem_g1)
        sem_c = (sem_c0, sem_c1)
        c = lax.axis_index("c")
        s = lax.axis_index("s")
        r0 = s * rps

        def zb(i, carry):
            zbuf[i, :] = jnp.zeros((L,), jnp.float32)
            return carry
        lax.fori_loop(0, WIN, zb, 0)

        def zc(k, carry):
            pltpu.sync_copy(zbuf, acc.at[pl.ds(r0 + k * WIN, WIN), :])
            return carry
        lax.fori_loop(0, nzc, zc, 0)
        plsc.subcore_barrier()

        if feat_split:
            w_first = s * win_per_worker
        else:
            w_first = (c * NS + s) * win_per_worker

        def stage(b, ss):
            w0 = w_first + ss * K
            pltpu.async_copy(srcw.at[pl.ds(w0, K)], idx_s[b], sem_s[b])
            pltpu.async_copy(dstw.at[pl.ds(w0, K)], idx_d[b], sem_d[b])

        def wait_stage(b):
            pltpu.make_async_copy(srcw.at[pl.ds(0, K)], idx_s[b], sem_s[b]).wait()
            pltpu.make_async_copy(dstw.at[pl.ds(0, K)], idx_d[b], sem_d[b]).wait()

        def fire(b):
            if feat_split:
                off = c * n_pad
                for j in range(K):
                    for i in range(WIN // L):
                        sl = pl.ds(i * L, L)
                        idx_s[b][j, sl] = idx_s[b][j, sl] + off
            for j in range(K):
                pltpu.async_copy(tbl.at[idx_s[b].at[j]], rows[b].at[j], sem_g[b])

        def wait_fire(b):
            for j in range(K):
                pltpu.make_async_copy(tbl.at[idx_s[b].at[j]], rows[b].at[j],
                                      sem_g[b]).wait()

        def scatter(b):
            for j in range(K):
                pltpu.async_copy(rows[b].at[j], acc.at[idx_d[b].at[j]],
                                 sem_c[b], add=True)
            for j in range(K):
                pltpu.make_async_copy(rows[b].at[j], acc.at[idx_d[b].at[j]],
                                      sem_c[b]).wait()

        stage(0, 0)
        wait_stage(0)
        fire(0)
        stage(1, 1)

        def pair(t, carry):
            ss0 = 2 * t
            for b in (0, 1):
                nb = 1 - b
                wait_stage(nb)
                fire(nb)
                wait_fire(b)
                scatter(b)
                stage(b, ss0 + b + 2)
            return carry
        lax.fori_loop(0, steps // 2, pair, 0)
        wait_fire(0)
        wait_stage(1)

        plsc.subcore_barrier()
        pltpu.sync_copy(acc.at[pl.ds(r0, rps), :],
                        out.at[pl.ds(c * n_pad + r0, rps), :])

    return pl.kernel(
        body,
        out_type=jax.ShapeDtypeStruct((NC * n_pad, L), jnp.float32),
        mesh=plsc.VectorSubcoreMesh(core_axis_name="c", subcore_axis_name="s"),
        compiler_params=pltpu.CompilerParams(use_tc_tiling_on_sc=False),
        scratch_types=[
            pltpu.VMEM_SHARED((n_pad, L), jnp.float32),
            pltpu.VMEM((K, WIN), jnp.int32),
            pltpu.VMEM((K, WIN), jnp.int32),
            pltpu.VMEM((K, WIN), jnp.int32),
            pltpu.VMEM((K, WIN), jnp.int32),
            pltpu.VMEM((K, WIN, L), jnp.float32),
            pltpu.VMEM((K, WIN, L), jnp.float32),
            pltpu.VMEM((WIN, L), jnp.float32),
        ] + [pltpu.SemaphoreType.DMA] * 8,
    )


def _elem_prop_kernel(n_pad, total_win, gather):
    rps = n_pad // NS
    nzc = rps // WIN
    win_per_worker = total_win // (NC * NS)
    steps = win_per_worker // K
    assert steps % 2 == 0

    def body(*refs):
        if gather:
            (tbl, srcw, dstw, out, acc,
             idx_s0, idx_s1, idx_d0, idx_d1, vals0, vals1, zbuf,
             sem_s0, sem_s1, sem_d0, sem_d1, sem_g0, sem_g1,
             sem_c0, sem_c1) = refs
            idx_s = (idx_s0, idx_s1)
            vals = (vals0, vals1)
            sem_s = (sem_s0, sem_s1)
            sem_g = (sem_g0, sem_g1)
        else:
            (dstw, out, acc, idx_d0, idx_d1, ones, zbuf,
             sem_d0, sem_d1, sem_c0, sem_c1) = refs
            vals = (ones, ones)
        idx_d = (idx_d0, idx_d1)
        sem_d = (sem_d0, sem_d1)
        sem_c = (sem_c0, sem_c1)
        c = lax.axis_index("c")
        s = lax.axis_index("s")
        r0 = s * rps

        def zb(i, carry):
            zbuf[pl.ds(i * L, L)] = jnp.zeros((L,), jnp.float32)
            return carry
        lax.fori_loop(0, WIN // L, zb, 0)

        def zc(k, carry):
            pltpu.sync_copy(zbuf, acc.at[pl.ds(r0 + k * WIN, WIN)])
            return carry
        lax.fori_loop(0, nzc, zc, 0)

        if not gather:
            for j in range(K):
                def ob(i, carry, j=j):
                    vals[0][j, pl.ds(i * L, L)] = jnp.ones((L,), jnp.float32)
                    return carry
                lax.fori_loop(0, WIN // L, ob, 0)
        plsc.subcore_barrier()

        w_first = (c * NS + s) * win_per_worker

        def stage(b, ss):
            w0 = w_first + ss * K
            if gather:
                pltpu.async_copy(srcw.at[pl.ds(w0, K)], idx_s[b], sem_s[b])
            pltpu.async_copy(dstw.at[pl.ds(w0, K)], idx_d[b], sem_d[b])

        def wait_stage(b):
            if gather:
                pltpu.make_async_copy(srcw.at[pl.ds(0, K)], idx_s[b],
                                      sem_s[b]).wait()
            pltpu.make_async_copy(dstw.at[pl.ds(0, K)], idx_d[b], sem_d[b]).wait()

        def fire(b):
            if gather:
                for j in range(K):
                    pltpu.async_copy(tbl.at[idx_s[b].at[j]], vals[b].at[j],
                                     sem_g[b])

        def wait_fire(b):
            if gather:
                for j in range(K):
                    pltpu.make_async_copy(tbl.at[idx_s[b].at[j]], vals[b].at[j],
                                          sem_g[b]).wait()

        def scatter(b):
            for j in range(K):
                pltpu.async_copy(vals[b].at[j], acc.at[idx_d[b].at[j]],
                                 sem_c[b], add=True)
            for j in range(K):
                pltpu.make_async_copy(vals[b].at[j], acc.at[idx_d[b].at[j]],
                                      sem_c[b]).wait()

        stage(0, 0)
        wait_stage(0)
        fire(0)
        stage(1, 1)

        def pair(t, carry):
            ss0 = 2 * t
            for b in (0, 1):
                nb = 1 - b
                wait_stage(nb)
                fire(nb)
                wait_fire(b)
                scatter(b)
                stage(b, ss0 + b + 2)
            return carry
        lax.fori_loop(0, steps // 2, pair, 0)
        wait_fire(0)
        wait_stage(1)

        plsc.subcore_barrier()
        pltpu.sync_copy(acc.at[pl.ds(r0, rps)],
                        out.at[pl.ds(c * n_pad + r0, rps)])

    scratch = [pltpu.VMEM_SHARED((n_pad,), jnp.float32)]
    if gather:
        scratch += [pltpu.VMEM((K, WIN), jnp.int32)] * 2
    scratch += [pltpu.VMEM((K, WIN), jnp.int32)] * 2
    if gather:
        scratch += [pltpu.VMEM((K, WIN), jnp.float32)] * 2
    else:
        scratch += [pltpu.VMEM((K, WIN), jnp.float32)]
    scratch += [pltpu.VMEM((WIN,), jnp.float32)]
    scratch += [pltpu.SemaphoreType.DMA] * (8 if gather else 4)

    return pl.kernel(
        body,
        out_type=jax.ShapeDtypeStruct((NC * n_pad,), jnp.float32),
        mesh=plsc.VectorSubcoreMesh(core_axis_name="c", subcore_axis_name="s"),
        compiler_params=pltpu.CompilerParams(use_tc_tiling_on_sc=False),
        scratch_types=scratch,
    )


def kernel(x, edge_index, W1, b1, W2, b2, W3, b3):
    n = x.shape[0]
    e = edge_index.shape[1]
    n_pad = NS * WIN * -(-n // (NS * WIN))
    step_edges = NC * NS * 2 * K * WIN
    e_pad = step_edges * -(-e // step_edges)
    total_win = e_pad // WIN
    s_rows = total_win + 2 * K
    br = n_pad // NS
    grid = (n_pad // br,)

    src = edge_index[0].astype(jnp.int32)
    dst = edge_index[1].astype(jnp.int32)
    npad_extra = s_rows * WIN - e
    pad = n + (jnp.arange(npad_extra, dtype=jnp.int32) % (n_pad - n))
    srcw = jnp.concatenate([src, pad]).reshape(s_rows, WIN)
    dstw = jnp.concatenate([dst, pad]).reshape(s_rows, WIN)

    f32 = jnp.float32
    sds = jax.ShapeDtypeStruct
    nb = n_pad // br

    halves = (pl.BlockSpec((br, 1), lambda i: (i, 0)),
              pl.BlockSpec((br, 1), lambda i: (nb + i, 0)))
    rhalf = (pl.BlockSpec((br, L), lambda i: (i, 0)),
             pl.BlockSpec((br, L), lambda i: (nb + i, 0)))

    degp = _elem_prop_kernel(n_pad, total_win, gather=False)(dstw)
    degp = degp.reshape(NC * n_pad, 1)

    def b1_body(pa_ref, pb_ref, x_ref, dis_ref, g0_ref):
        deg = pa_ref[...] + pb_ref[...] + 1.0
        dis = lax.rsqrt(deg)
        dis_ref[...] = dis
        g0_ref[...] = x_ref[...] * dis

    dis, g0 = pl.pallas_call(
        b1_body,
        grid=grid,
        in_specs=[*halves, pl.BlockSpec((br, L), lambda i: (i, 0))],
        out_specs=[
            pl.BlockSpec((br, 1), lambda i: (i, 0)),
            pl.BlockSpec((br, L), lambda i: (i, 0)),
        ],
        out_shape=[sds((n_pad, 1), f32), sds((n_pad, L), f32)],
    )(degp, degp, x)

    p0 = _row_prop_kernel(n_pad, total_win, feat_split=False)(g0, srcw, dstw)

    def b2_body(dis_ref, pa_ref, pb_ref, g0_ref, w1_ref, b1_ref, out_ref):
        sel = pl.program_id(0) >= nb
        dis = dis_ref[...]
        prop0 = dis * (pa_ref[...] + pb_ref[...] + g0_ref[...])
        h1 = jnp.dot(prop0, w1_ref[...], preferred_element_type=jnp.float32)
        h1 = jnp.maximum(h1 + b1_ref[...], 0.0)
        g1 = dis * h1
        out_ref[...] = jnp.where(sel, g1[:, L:], g1[:, :L])

    g1h = pl.pallas_call(
        b2_body,
        grid=(2 * nb,),
        in_specs=[
            pl.BlockSpec((br, 1), lambda i: (i % nb, 0)),
            pl.BlockSpec((br, L), lambda i: (i % nb, 0)),
            pl.BlockSpec((br, L), lambda i: (nb + i % nb, 0)),
            pl.BlockSpec((br, L), lambda i: (i % nb, 0)),
            pl.BlockSpec((L, 2 * L), lambda i: (0, 0)),
            pl.BlockSpec((1, 2 * L), lambda i: (0, 0)),
        ],
        out_specs=pl.BlockSpec((br, L), lambda i: (i, 0)),
        out_shape=sds((NC * n_pad, L), f32),
    )(dis, p0, p0, g0, W1, b1.reshape(1, 2 * L))

    p1 = _row_prop_kernel(n_pad, total_win, feat_split=True)(g1h, srcw, dstw)

    def b3_body(dis_ref, pa_ref, pb_ref, ga_ref, gb_ref, w2_ref, b2_ref,
                w3_ref, out_ref):
        dis = dis_ref[...]
        h32 = jnp.concatenate([dis * (pa_ref[...] + ga_ref[...]),
                               dis * (pb_ref[...] + gb_ref[...])], axis=1)
        h2 = jnp.dot(h32, w2_ref[...], preferred_element_type=jnp.float32)
        h2 = jnp.maximum(h2 + b2_ref[...], 0.0)
        s = jnp.dot(h2, w3_ref[...], preferred_element_type=jnp.float32)
        out_ref[...] = dis * s

    g2 = pl.pallas_call(
        b3_body,
        grid=grid,
        in_specs=[
            pl.BlockSpec((br, 1), lambda i: (i, 0)),
            *rhalf,
            *rhalf,
            pl.BlockSpec((2 * L, 4 * L), lambda i: (0, 0)),
            pl.BlockSpec((1, 4 * L), lambda i: (0, 0)),
            pl.BlockSpec((4 * L, 1), lambda i: (0, 0)),
        ],
        out_specs=pl.BlockSpec((br, 1), lambda i: (i, 0)),
        out_shape=sds((n_pad, 1), f32),
    )(dis, p1, p1, g1h, g1h, W2, b2.reshape(1, 4 * L), W3)

    p2 = _elem_prop_kernel(n_pad, total_win, gather=True)(
        g2.reshape(n_pad), srcw, dstw)
    p2 = p2.reshape(NC * n_pad, 1)

    def b4_body(dis_ref, pa_ref, pb_ref, g2_ref, b3_ref, out_ref):
        t = pa_ref[...] + pb_ref[...] + g2_ref[...]
        t = dis_ref[...] * t + b3_ref[0, 0]
        out_ref[...] = jax.nn.sigmoid(t)

    out = pl.pallas_call(
        b4_body,
        grid=grid,
        in_specs=[
            pl.BlockSpec((br, 1), lambda i: (i, 0)),
            *halves,
            pl.BlockSpec((br, 1), lambda i: (i, 0)),
            pl.BlockSpec((1, 1), lambda i: (0, 0)),
        ],
        out_specs=pl.BlockSpec((br, 1), lambda i: (i, 0)),
        out_shape=sds((n_pad, 1), f32),
    )(dis, p2, p2, g2, b3.reshape(1, 1))

    return out[:n]


# speedup vs baseline: 63.9197x; 1.0211x over previous
import jax
import jax.numpy as jnp
from jax import lax
from jax.experimental import pallas as pl
from jax.experimental.pallas import tpu as pltpu
from jax.experimental.pallas import tpu_sc as plsc

NC = 2
NS = 16
L = 16
WIN = 128
K = 6
KE = 12


def _row_prop_kernel(n_pad, total_win, feat_split):
    rps = n_pad // NS
    nzc = rps // WIN
    if feat_split:
        win_per_worker = total_win // NS
    else:
        win_per_worker = total_win // (NC * NS)
    steps = win_per_worker // K
    assert steps % 2 == 0

    def body(tbl, srcw, dstw, out, acc,
             idx_s0, idx_s1, idx_d0, idx_d1, rows0, rows1, zbuf,
             sem_s0, sem_s1, sem_d0, sem_d1, sem_g0, sem_g1, sem_c0, sem_c1):
        idx_s = (idx_s0, idx_s1)
        idx_d = (idx_d0, idx_d1)
        rows = (rows0, rows1)
        sem_s = (sem_s0, sem_s1)
        sem_d = (sem_d0, sem_d1)
        sem_g = (sem_g0, sem_g1)
        sem_c = (sem_c0, sem_c1)
        c = lax.axis_index("c")
        s = lax.axis_index("s")
        r0 = s * rps

        def zb(i, carry):
            zbuf[i, :] = jnp.zeros((L,), jnp.float32)
            return carry
        lax.fori_loop(0, WIN, zb, 0)

        def zc(k, carry):
            pltpu.sync_copy(zbuf, acc.at[pl.ds(r0 + k * WIN, WIN), :])
            return carry
        lax.fori_loop(0, nzc, zc, 0)
        plsc.subcore_barrier()

        if feat_split:
            w_first = s * win_per_worker
        else:
            w_first = (c * NS + s) * win_per_worker

        def stage(b, ss):
            w0 = w_first + ss * K
            pltpu.async_copy(srcw.at[pl.ds(w0, K)], idx_s[b], sem_s[b])
            pltpu.async_copy(dstw.at[pl.ds(w0, K)], idx_d[b], sem_d[b])

        def wait_stage(b):
            pltpu.make_async_copy(srcw.at[pl.ds(0, K)], idx_s[b], sem_s[b]).wait()
            pltpu.make_async_copy(dstw.at[pl.ds(0, K)], idx_d[b], sem_d[b]).wait()

        def fire(b):
            if feat_split:
                off = c * n_pad
                for j in range(K):
                    for i in range(WIN // L):
                        sl = pl.ds(i * L, L)
                        idx_s[b][j, sl] = idx_s[b][j, sl] + off
            for j in range(K):
                pltpu.async_copy(tbl.at[idx_s[b].at[j]], rows[b].at[j], sem_g[b])

        def wait_fire(b):
            for j in range(K):
                pltpu.make_async_copy(tbl.at[idx_s[b].at[j]], rows[b].at[j],
                                      sem_g[b]).wait()

        def scatter(b):
            for j in range(K):
                pltpu.async_copy(rows[b].at[j], acc.at[idx_d[b].at[j]],
                                 sem_c[b], add=True)
            for j in range(K):
                pltpu.make_async_copy(rows[b].at[j], acc.at[idx_d[b].at[j]],
                                      sem_c[b]).wait()

        stage(0, 0)
        wait_stage(0)
        fire(0)
        stage(1, 1)

        def pair(t, carry):
            ss0 = 2 * t
            for b in (0, 1):
                nb = 1 - b
                wait_stage(nb)
                fire(nb)
                wait_fire(b)
                scatter(b)
                stage(b, ss0 + b + 2)
            return carry
        lax.fori_loop(0, steps // 2, pair, 0)
        wait_fire(0)
        wait_stage(1)

        plsc.subcore_barrier()
        pltpu.sync_copy(acc.at[pl.ds(r0, rps), :],
                        out.at[pl.ds(c * n_pad + r0, rps), :])

    return pl.kernel(
        body,
        out_type=jax.ShapeDtypeStruct((NC * n_pad, L), jnp.float32),
        mesh=plsc.VectorSubcoreMesh(core_axis_name="c", subcore_axis_name="s"),
        compiler_params=pltpu.CompilerParams(use_tc_tiling_on_sc=False),
        scratch_types=[
            pltpu.VMEM_SHARED((n_pad, L), jnp.float32),
            pltpu.VMEM((K, WIN), jnp.int32),
            pltpu.VMEM((K, WIN), jnp.int32),
            pltpu.VMEM((K, WIN), jnp.int32),
            pltpu.VMEM((K, WIN), jnp.int32),
            pltpu.VMEM((K, WIN, L), jnp.float32),
            pltpu.VMEM((K, WIN, L), jnp.float32),
            pltpu.VMEM((WIN, L), jnp.float32),
        ] + [pltpu.SemaphoreType.DMA] * 8,
    )


def _elem_prop_kernel(n_pad, total_win, gather):
    K = KE
    rps = n_pad // NS
    nzc = rps // WIN
    win_per_worker = total_win // (NC * NS)
    steps = win_per_worker // K
    assert steps % 2 == 0

    def body(*refs):
        if gather:
            (tbl, srcw, dstw, out, acc,
             idx_s0, idx_s1, idx_d0, idx_d1, vals0, vals1, zbuf,
             sem_s0, sem_s1, sem_d0, sem_d1, sem_g0, sem_g1,
             sem_c0, sem_c1) = refs
            idx_s = (idx_s0, idx_s1)
            vals = (vals0, vals1)
            sem_s = (sem_s0, sem_s1)
            sem_g = (sem_g0, sem_g1)
        else:
            (dstw, out, acc, idx_d0, idx_d1, ones, zbuf,
             sem_d0, sem_d1, sem_c0, sem_c1) = refs
            vals = (ones, ones)
        idx_d = (idx_d0, idx_d1)
        sem_d = (sem_d0, sem_d1)
        sem_c = (sem_c0, sem_c1)
        c = lax.axis_index("c")
        s = lax.axis_index("s")
        r0 = s * rps

        def zb(i, carry):
            zbuf[pl.ds(i * L, L)] = jnp.zeros((L,), jnp.float32)
            return carry
        lax.fori_loop(0, WIN // L, zb, 0)

        def zc(k, carry):
            pltpu.sync_copy(zbuf, acc.at[pl.ds(r0 + k * WIN, WIN)])
            return carry
        lax.fori_loop(0, nzc, zc, 0)

        if not gather:
            for j in range(K):
                def ob(i, carry, j=j):
                    vals[0][j, pl.ds(i * L, L)] = jnp.ones((L,), jnp.float32)
                    return carry
                lax.fori_loop(0, WIN // L, ob, 0)
        plsc.subcore_barrier()

        w_first = (c * NS + s) * win_per_worker

        def stage(b, ss):
            w0 = w_first + ss * K
            if gather:
                pltpu.async_copy(srcw.at[pl.ds(w0, K)], idx_s[b], sem_s[b])
            pltpu.async_copy(dstw.at[pl.ds(w0, K)], idx_d[b], sem_d[b])

        def wait_stage(b):
            if gather:
                pltpu.make_async_copy(srcw.at[pl.ds(0, K)], idx_s[b],
                                      sem_s[b]).wait()
            pltpu.make_async_copy(dstw.at[pl.ds(0, K)], idx_d[b], sem_d[b]).wait()

        def fire(b):
            if gather:
                for j in range(K):
                    pltpu.async_copy(tbl.at[idx_s[b].at[j]], vals[b].at[j],
                                     sem_g[b])

        def wait_fire(b):
            if gather:
                for j in range(K):
                    pltpu.make_async_copy(tbl.at[idx_s[b].at[j]], vals[b].at[j],
                                          sem_g[b]).wait()

        def scatter(b):
            for j in range(K):
                pltpu.async_copy(vals[b].at[j], acc.at[idx_d[b].at[j]],
                                 sem_c[b], add=True)
            for j in range(K):
                pltpu.make_async_copy(vals[b].at[j], acc.at[idx_d[b].at[j]],
                                      sem_c[b]).wait()

        stage(0, 0)
        wait_stage(0)
        fire(0)
        stage(1, 1)

        def pair(t, carry):
            ss0 = 2 * t
            for b in (0, 1):
                nb = 1 - b
                wait_stage(nb)
                fire(nb)
                wait_fire(b)
                scatter(b)
                stage(b, ss0 + b + 2)
            return carry
        lax.fori_loop(0, steps // 2, pair, 0)
        wait_fire(0)
        wait_stage(1)

        plsc.subcore_barrier()
        pltpu.sync_copy(acc.at[pl.ds(r0, rps)],
                        out.at[pl.ds(c * n_pad + r0, rps)])

    scratch = [pltpu.VMEM_SHARED((n_pad,), jnp.float32)]
    if gather:
        scratch += [pltpu.VMEM((K, WIN), jnp.int32)] * 2
    scratch += [pltpu.VMEM((K, WIN), jnp.int32)] * 2
    if gather:
        scratch += [pltpu.VMEM((K, WIN), jnp.float32)] * 2
    else:
        scratch += [pltpu.VMEM((K, WIN), jnp.float32)]
    scratch += [pltpu.VMEM((WIN,), jnp.float32)]
    scratch += [pltpu.SemaphoreType.DMA] * (8 if gather else 4)

    return pl.kernel(
        body,
        out_type=jax.ShapeDtypeStruct((NC * n_pad,), jnp.float32),
        mesh=plsc.VectorSubcoreMesh(core_axis_name="c", subcore_axis_name="s"),
        compiler_params=pltpu.CompilerParams(use_tc_tiling_on_sc=False),
        scratch_types=scratch,
    )


def kernel(x, edge_index, W1, b1, W2, b2, W3, b3):
    n = x.shape[0]
    e = edge_index.shape[1]
    n_pad = NS * WIN * -(-n // (NS * WIN))
    step_edges = NC * NS * 2 * max(K, KE) * WIN
    e_pad = step_edges * -(-e // step_edges)
    total_win = e_pad // WIN
    s_rows = total_win + 2 * max(K, KE)
    br = n_pad // NS
    grid = (n_pad // br,)

    src = edge_index[0].astype(jnp.int32)
    dst = edge_index[1].astype(jnp.int32)
    npad_extra = s_rows * WIN - e
    pad = n + (jnp.arange(npad_extra, dtype=jnp.int32) % (n_pad - n))
    srcw = jnp.concatenate([src, pad]).reshape(s_rows, WIN)
    dstw = jnp.concatenate([dst, pad]).reshape(s_rows, WIN)

    f32 = jnp.float32
    sds = jax.ShapeDtypeStruct
    nb = n_pad // br

    halves = (pl.BlockSpec((br, 1), lambda i: (i, 0)),
              pl.BlockSpec((br, 1), lambda i: (nb + i, 0)))
    rhalf = (pl.BlockSpec((br, L), lambda i: (i, 0)),
             pl.BlockSpec((br, L), lambda i: (nb + i, 0)))

    degp = _elem_prop_kernel(n_pad, total_win, gather=False)(dstw)
    degp = degp.reshape(NC * n_pad, 1)

    def b1_body(pa_ref, pb_ref, x_ref, dis_ref, g0_ref):
        deg = pa_ref[...] + pb_ref[...] + 1.0
        dis = lax.rsqrt(deg)
        dis_ref[...] = dis
        g0_ref[...] = x_ref[...] * dis

    dis, g0 = pl.pallas_call(
        b1_body,
        grid=grid,
        in_specs=[*halves, pl.BlockSpec((br, L), lambda i: (i, 0))],
        out_specs=[
            pl.BlockSpec((br, 1), lambda i: (i, 0)),
            pl.BlockSpec((br, L), lambda i: (i, 0)),
        ],
        out_shape=[sds((n_pad, 1), f32), sds((n_pad, L), f32)],
    )(degp, degp, x)

    p0 = _row_prop_kernel(n_pad, total_win, feat_split=False)(g0, srcw, dstw)

    def b2_body(dis_ref, pa_ref, pb_ref, g0_ref, w1_ref, b1_ref, out_ref):
        sel = pl.program_id(0) >= nb
        dis = dis_ref[...]
        prop0 = dis * (pa_ref[...] + pb_ref[...] + g0_ref[...])
        h1 = jnp.dot(prop0, w1_ref[...], preferred_element_type=jnp.float32)
        h1 = jnp.maximum(h1 + b1_ref[...], 0.0)
        g1 = dis * h1
        out_ref[...] = jnp.where(sel, g1[:, L:], g1[:, :L])

    g1h = pl.pallas_call(
        b2_body,
        grid=(2 * nb,),
        in_specs=[
            pl.BlockSpec((br, 1), lambda i: (i % nb, 0)),
            pl.BlockSpec((br, L), lambda i: (i % nb, 0)),
            pl.BlockSpec((br, L), lambda i: (nb + i % nb, 0)),
            pl.BlockSpec((br, L), lambda i: (i % nb, 0)),
            pl.BlockSpec((L, 2 * L), lambda i: (0, 0)),
            pl.BlockSpec((1, 2 * L), lambda i: (0, 0)),
        ],
        out_specs=pl.BlockSpec((br, L), lambda i: (i, 0)),
        out_shape=sds((NC * n_pad, L), f32),
    )(dis, p0, p0, g0, W1, b1.reshape(1, 2 * L))

    p1 = _row_prop_kernel(n_pad, total_win, feat_split=True)(g1h, srcw, dstw)

    def b3_body(dis_ref, pa_ref, pb_ref, ga_ref, gb_ref, w2_ref, b2_ref,
                w3_ref, out_ref):
        dis = dis_ref[...]
        h32 = jnp.concatenate([dis * (pa_ref[...] + ga_ref[...]),
                               dis * (pb_ref[...] + gb_ref[...])], axis=1)
        h2 = jnp.dot(h32, w2_ref[...], preferred_element_type=jnp.float32)
        h2 = jnp.maximum(h2 + b2_ref[...], 0.0)
        s = jnp.dot(h2, w3_ref[...], preferred_element_type=jnp.float32)
        out_ref[...] = dis * s

    g2 = pl.pallas_call(
        b3_body,
        grid=grid,
        in_specs=[
            pl.BlockSpec((br, 1), lambda i: (i, 0)),
            *rhalf,
            *rhalf,
            pl.BlockSpec((2 * L, 4 * L), lambda i: (0, 0)),
            pl.BlockSpec((1, 4 * L), lambda i: (0, 0)),
            pl.BlockSpec((4 * L, 1), lambda i: (0, 0)),
        ],
        out_specs=pl.BlockSpec((br, 1), lambda i: (i, 0)),
        out_shape=sds((n_pad, 1), f32),
    )(dis, p1, p1, g1h, g1h, W2, b2.reshape(1, 4 * L), W3)

    p2 = _elem_prop_kernel(n_pad, total_win, gather=True)(
        g2.reshape(n_pad), srcw, dstw)
    p2 = p2.reshape(NC * n_pad, 1)

    def b4_body(dis_ref, pa_ref, pb_ref, g2_ref, b3_ref, out_ref):
        t = pa_ref[...] + pb_ref[...] + g2_ref[...]
        t = dis_ref[...] * t + b3_ref[0, 0]
        out_ref[...] = jax.nn.sigmoid(t)

    out = pl.pallas_call(
        b4_body,
        grid=grid,
        in_specs=[
            pl.BlockSpec((br, 1), lambda i: (i, 0)),
            *halves,
            pl.BlockSpec((br, 1), lambda i: (i, 0)),
            pl.BlockSpec((1, 1), lambda i: (0, 0)),
        ],
        out_specs=pl.BlockSpec((br, 1), lambda i: (i, 0)),
        out_shape=sds((n_pad, 1), f32),
    )(dis, p2, p2, g2, b3.reshape(1, 1))

    return out[:n]
